# Initial kernel scaffold; baseline (speedup 1.0000x reference)
import jax
import jax.numpy as jnp
from jax.experimental import pallas as pl


def kernel(x, edge_index, W, b, gamma, beta):
    raise NotImplementedError("write your pallas kernel here")


# speedup vs baseline: 15.7106x; 15.7106x over previous
import functools

import jax
import jax.numpy as jnp
from jax import lax
from jax.experimental import pallas as pl
from jax.experimental.pallas import tpu as pltpu
from jax.experimental.pallas import tpu_sc as plsc

NW = 32
CHUNK = 80


def _sc_mesh():
    return plsc.VectorSubcoreMesh(core_axis_name="c", subcore_axis_name="s")


def _make_deg_kernel(E, N_PAD, stripe, chunks, per_tile):
    @functools.partial(
        pl.kernel,
        out_type=jax.ShapeDtypeStruct((2 * N_PAD,), jnp.float32),
        mesh=_sc_mesh(),
        scratch_types=[
            pltpu.VMEM((CHUNK,), jnp.int32),
            pltpu.VMEM((CHUNK,), jnp.int32),
            pltpu.VMEM((CHUNK,), jnp.int32),
            pltpu.VMEM((CHUNK,), jnp.float32),
            pltpu.VMEM((stripe,), jnp.float32),
            pltpu.VMEM_SHARED((N_PAD,), jnp.float32),
        ],
    )
    def deg_kernel(row_hbm, col_hbm, out_hbm,
                   rowv, colv, idxv, onesv, stagev, acc_sh):
        c = lax.axis_index("c")
        s = lax.axis_index("s")
        wid = s * 2 + c
        for i in range(CHUNK // 16):
            onesv[pl.ds(i * 16, 16)] = jnp.ones((16,), jnp.float32)

        def zfill(j, carry):
            stagev[pl.ds(j * 16, 16)] = jnp.zeros((16,), jnp.float32)
            return carry

        lax.fori_loop(0, stripe // 16, zfill, 0)
        pltpu.sync_copy(stagev, acc_sh.at[pl.ds(s * stripe, stripe)])
        plsc.subcore_barrier()

        def body(j, carry):
            base = wid * per_tile + j * CHUNK
            pltpu.sync_copy(row_hbm.at[pl.ds(base, CHUNK)], rowv)
            pltpu.sync_copy(col_hbm.at[pl.ds(base, CHUNK)], colv)
            for i in range(CHUNK // 16):
                sl = pl.ds(i * 16, 16)
                r = rowv[sl]
                cc = colv[sl]
                idxv[sl] = jnp.where(r == cc, N_PAD - 8, r)
            pltpu.sync_copy(onesv, acc_sh.at[idxv], add=True)
            return carry

        lax.fori_loop(0, chunks, body, 0)
        plsc.subcore_barrier()
        pltpu.sync_copy(acc_sh.at[pl.ds(s * stripe, stripe)], stagev)
        pltpu.sync_copy(stagev,
                        out_hbm.at[pl.ds(c * N_PAD + s * stripe, stripe)])

    return deg_kernel


def _make_agg_kernel(E, N_PAD, D, stripe, chunks, per_tile):
    @functools.partial(
        pl.kernel,
        out_type=jax.ShapeDtypeStruct((2, N_PAD, D), jnp.float32),
        mesh=_sc_mesh(),
        scratch_types=[
            pltpu.VMEM((CHUNK,), jnp.int32),
            pltpu.VMEM((CHUNK,), jnp.int32),
            pltpu.VMEM((CHUNK,), jnp.int32),
            pltpu.VMEM((CHUNK, D), jnp.float32),
            pltpu.SemaphoreType.DMA,
            pltpu.VMEM_SHARED((N_PAD, D), jnp.float32),
        ],
    )
    def agg_kernel(row_hbm, col_hbm, y_hbm, out_hbm,
                   rowv, colv, idxv, rowsv, sem, acc_sh):
        c = lax.axis_index("c")
        s = lax.axis_index("s")
        wid = s * 2 + c

        def zrow(i, carry):
            for k in range(D // 16):
                rowsv[i, pl.ds(k * 16, 16)] = jnp.zeros((16,), jnp.float32)
            return carry

        lax.fori_loop(0, CHUNK, zrow, 0)

        def zcp(t, carry):
            pltpu.sync_copy(rowsv,
                            acc_sh.at[pl.ds(s * stripe + t * CHUNK, CHUNK), :])
            return carry

        lax.fori_loop(0, stripe // CHUNK, zcp, 0)
        plsc.subcore_barrier()

        def body(j, carry):
            base = wid * per_tile + j * CHUNK
            pltpu.sync_copy(row_hbm.at[pl.ds(base, CHUNK)], rowv)
            pltpu.sync_copy(col_hbm.at[pl.ds(base, CHUNK)], colv)
            cp = pltpu.async_copy(y_hbm.at[colv], rowsv, sem)
            for i in range(CHUNK // 16):
                sl = pl.ds(i * 16, 16)
                r = rowv[sl]
                cc = colv[sl]
                idxv[sl] = jnp.where(r == cc, N_PAD - 8, r)
            cp.wait()
            pltpu.sync_copy(rowsv, acc_sh.at[idxv], add=True)
            return carry

        lax.fori_loop(0, chunks, body, 0)
        plsc.subcore_barrier()

        def ocp(t, carry):
            base2 = s * stripe + t * CHUNK
            pltpu.sync_copy(acc_sh.at[pl.ds(base2, CHUNK), :], rowsv)
            pltpu.sync_copy(rowsv, out_hbm.at[c, pl.ds(base2, CHUNK), :])
            return carry

        lax.fori_loop(0, stripe // CHUNK, ocp, 0)

    return agg_kernel


def _y_body(x_ref, degp_ref, y_ref):
    d = degp_ref[:, 0] + degp_ref[:, 1] + 1.0
    dis = lax.rsqrt(d)
    y_ref[...] = x_ref[...] * dis[:, None]


def _m_body(accp_ref, y_ref, degp_ref, w_ref, b_ref, h_ref, stats_ref):
    i = pl.program_id(0)
    d = degp_ref[:, 0] + degp_ref[:, 1] + 1.0
    dis = lax.rsqrt(d)
    pre = (accp_ref[0] + accp_ref[1] + y_ref[...]) * dis[:, None]
    h = lax.dot_general(pre, w_ref[...], (((1,), (1,)), ((), ())),
                        preferred_element_type=jnp.float32)
    h = jnp.maximum(h + b_ref[...], 0.0)
    h_ref[...] = h

    @pl.when(i == 0)
    def _():
        stats_ref[...] = jnp.zeros_like(stats_ref)

    stats_ref[0:1, :] += jnp.sum(h, axis=0, keepdims=True)
    stats_ref[1:2, :] += jnp.sum(h * h, axis=0, keepdims=True)


def _make_bn_body(N):
    def bn_body(h_ref, stats_ref, gamma_ref, beta_ref, out_ref):
        inv_n = 1.0 / N
        mean = stats_ref[0:1, :] * inv_n
        ex2 = stats_ref[1:2, :] * inv_n
        var = ex2 - mean * mean
        inv = lax.rsqrt(var + 1e-5)
        out_ref[...] = (h_ref[...] - mean) * inv * gamma_ref[...] + beta_ref[...]
    return bn_body


def kernel(x, edge_index, W, b, gamma, beta):
    N, D = x.shape
    E = edge_index.shape[1]
    N_PAD = ((N + 8) + 255) // 256 * 256
    stripe = N_PAD // 16
    per_tile = E // NW
    chunks = per_tile // CHUNK

    row = edge_index[0]
    col = edge_index[1]

    deg_p = _make_deg_kernel(E, N_PAD, stripe, chunks, per_tile)(row, col)
    degp_t = deg_p.reshape(2, N_PAD).T

    BN = 1000
    grid = (N // BN,)
    y = pl.pallas_call(
        _y_body,
        grid=grid,
        in_specs=[
            pl.BlockSpec((BN, D), lambda i: (i, 0)),
            pl.BlockSpec((BN, 2), lambda i: (i, 0)),
        ],
        out_specs=pl.BlockSpec((BN, D), lambda i: (i, 0)),
        out_shape=jax.ShapeDtypeStruct((N, D), jnp.float32),
    )(x, degp_t)

    acc_p = _make_agg_kernel(E, N_PAD, D, stripe, chunks, per_tile)(
        row, col, y)

    b2 = b.reshape(1, D)
    h, stats = pl.pallas_call(
        _m_body,
        grid=grid,
        in_specs=[
            pl.BlockSpec((2, BN, D), lambda i: (0, i, 0)),
            pl.BlockSpec((BN, D), lambda i: (i, 0)),
            pl.BlockSpec((BN, 2), lambda i: (i, 0)),
            pl.BlockSpec((D, D), lambda i: (0, 0)),
            pl.BlockSpec((1, D), lambda i: (0, 0)),
        ],
        out_specs=[
            pl.BlockSpec((BN, D), lambda i: (i, 0)),
            pl.BlockSpec((8, D), lambda i: (0, 0)),
        ],
        out_shape=[
            jax.ShapeDtypeStruct((N, D), jnp.float32),
            jax.ShapeDtypeStruct((8, D), jnp.float32),
        ],
    )(acc_p, y, degp_t, W, b2)

    out = pl.pallas_call(
        _make_bn_body(N),
        grid=grid,
        in_specs=[
            pl.BlockSpec((BN, D), lambda i: (i, 0)),
            pl.BlockSpec((8, D), lambda i: (0, 0)),
            pl.BlockSpec((1, D), lambda i: (0, 0)),
            pl.BlockSpec((1, D), lambda i: (0, 0)),
        ],
        out_specs=pl.BlockSpec((BN, D), lambda i: (i, 0)),
        out_shape=jax.ShapeDtypeStruct((N, D), jnp.float32),
    )(h, stats, gamma.reshape(1, D), beta.reshape(1, D))
    return out
